# baseline (speedup 1.0000x reference)
import jax
import jax.numpy as jnp
from jax.experimental import pallas as pl
from jax.experimental.pallas import tpu as pltpu

_B, _N, _D = 4, 4096, 3
_K = 8
_NB = 1024


def _chamfer_body(xw_ref, x_ref, y_ref, out_ref):
    y = y_ref[0]
    ysq = jnp.sum(y * y, axis=0, keepdims=True)
    hi = ysq.astype(jnp.bfloat16)
    r1 = ysq - hi.astype(jnp.float32)
    mid = r1.astype(jnp.bfloat16)
    lo = (r1 - mid.astype(jnp.float32)).astype(jnp.bfloat16)
    w = jnp.concatenate(
        [y.astype(jnp.bfloat16), hi, mid, lo,
         jnp.zeros((_K - 2 * _D, _N), jnp.bfloat16)], axis=0
    )

    xw = xw_ref[0]
    t = jax.lax.dot_general(
        xw, w, (((1,), (0,)), ((), ())),
        preferred_element_type=jnp.float32,
    )
    m = jnp.min(t, axis=1, keepdims=True)

    x = x_ref[0]
    xsq = jnp.sum(x * x, axis=1, keepdims=True)
    bsum = jnp.sum(jnp.maximum(m + xsq, 0.0))
    out_ref[...] = jnp.full((8, 128), bsum, jnp.float32)


def kernel(v, v_pred):
    yT = jnp.transpose(v, (0, 2, 1))
    ones = jnp.ones((_B, _N, _D), jnp.bfloat16)
    zeros = jnp.zeros((_B, _N, _K - 2 * _D), jnp.bfloat16)
    xw = jnp.concatenate(
        [(v_pred * -2.0).astype(jnp.bfloat16), ones, zeros], axis=2
    )
    out = pl.pallas_call(
        _chamfer_body,
        grid=(_B, _N // _NB),
        in_specs=[
            pl.BlockSpec((1, _NB, _K), lambda b, n: (b, n, 0)),
            pl.BlockSpec((1, _NB, _D), lambda b, n: (b, n, 0)),
            pl.BlockSpec((1, _D, _N), lambda b, n: (b, 0, 0)),
        ],
        out_specs=pl.BlockSpec((8, 128), lambda b, n: (b, n)),
        out_shape=jax.ShapeDtypeStruct(
            (_B * 8, (_N // _NB) * 128), jnp.float32
        ),
        compiler_params=pltpu.CompilerParams(
            dimension_semantics=("parallel", "parallel"),
        ),
    )(xw, v_pred, yT)
    return jnp.sum(out) * (1.0 / (_B * _N * 8 * 128))


# speedup vs baseline: 1.1233x; 1.1233x over previous
import jax
import jax.numpy as jnp
from jax.experimental import pallas as pl
from jax.experimental.pallas import tpu as pltpu

_B, _N, _D = 4, 4096, 3
_K = 8
_NB = 1024


def _chamfer_body(x_ref, y_ref, out_ref):
    y = y_ref[0]
    ysq = jnp.sum(y * y, axis=0, keepdims=True)
    hi = ysq.astype(jnp.bfloat16)
    r1 = ysq - hi.astype(jnp.float32)
    mid = r1.astype(jnp.bfloat16)
    lo = (r1 - mid.astype(jnp.float32)).astype(jnp.bfloat16)
    w = jnp.concatenate(
        [y.astype(jnp.bfloat16), hi, mid, lo,
         jnp.zeros((_K - 2 * _D, _N), jnp.bfloat16)], axis=0
    )

    x = x_ref[0]
    xw = jnp.concatenate(
        [(x * -2.0).astype(jnp.bfloat16),
         jnp.ones((_NB, _D), jnp.bfloat16),
         jnp.zeros((_NB, _K - 2 * _D), jnp.bfloat16)], axis=1
    )

    t = jax.lax.dot_general(
        xw, w, (((1,), (0,)), ((), ())),
        preferred_element_type=jnp.float32,
    )
    m = jnp.min(t, axis=1, keepdims=True)

    xsq = jnp.sum(x * x, axis=1, keepdims=True)
    bsum = jnp.sum(jnp.maximum(m + xsq, 0.0))
    out_ref[...] = jnp.full((8, 128), bsum, jnp.float32)


def kernel(v, v_pred):
    yT = jnp.transpose(v, (0, 2, 1))
    out = pl.pallas_call(
        _chamfer_body,
        grid=(_B, _N // _NB),
        in_specs=[
            pl.BlockSpec((1, _NB, _D), lambda b, n: (b, n, 0)),
            pl.BlockSpec((1, _D, _N), lambda b, n: (b, 0, 0)),
        ],
        out_specs=pl.BlockSpec((8, 128), lambda b, n: (b, n)),
        out_shape=jax.ShapeDtypeStruct(
            (_B * 8, (_N // _NB) * 128), jnp.float32
        ),
        compiler_params=pltpu.CompilerParams(
            dimension_semantics=("parallel", "parallel"),
        ),
    )(v_pred, yT)
    return jnp.sum(out) * (1.0 / (_B * _N * 8 * 128))
